# Initial kernel scaffold; baseline (speedup 1.0000x reference)
import jax
import jax.numpy as jnp
from jax.experimental import pallas as pl


def kernel(x, W_lin, b_lin, W_gat, att_src, att_dst, bias_gat):
    raise NotImplementedError("write your pallas kernel here")


# speedup vs baseline: 118.7115x; 118.7115x over previous
import jax
import jax.numpy as jnp
from jax.experimental import pallas as pl

_B, _N, _IN_DIM, _FEA, _H = 8, 64, 512, 256, 8
_NEG_SLOPE = 0.2


def _gat_proto_kernel(x_ref, wlin_ref, blin_ref, wgat_ref, asrc_ref,
                      adst_ref, bias_ref, out_ref):
    h = jnp.dot(x_ref[...], wlin_ref[...],
                preferred_element_type=jnp.float32) + blin_ref[...]
    acc = jnp.zeros((1, _FEA), dtype=jnp.float32)
    for hh in range(_H):
        Wh = wgat_ref[:, hh * _FEA:(hh + 1) * _FEA]
        vs = (Wh * asrc_ref[hh:hh + 1, :]).sum(axis=1, keepdims=True)
        vd = (Wh * adst_ref[hh:hh + 1, :]).sum(axis=1, keepdims=True)
        a_s = jnp.dot(h, vs, preferred_element_type=jnp.float32)
        a_d = jnp.dot(h, vd, preferred_element_type=jnp.float32)
        s = a_d + a_s.T
        s = jnp.where(s >= 0, s, _NEG_SLOPE * s)
        m = jnp.max(s, axis=1, keepdims=True)
        ex = jnp.exp(s - m)
        denom = jnp.sum(ex, axis=1, keepdims=True) + 1e-16
        w = jnp.sum(ex / denom, axis=0, keepdims=True)
        g = jnp.dot(w, h, preferred_element_type=jnp.float32)
        acc = acc + jnp.dot(g, Wh, preferred_element_type=jnp.float32)
    out_ref[0] = acc * (1.0 / (_N * _H)) + bias_ref[...]


def kernel(x, W_lin, b_lin, W_gat, att_src, att_dst, bias_gat):
    x2 = x.reshape(_B * _N, _IN_DIM)
    blin2 = b_lin.reshape(1, _FEA)
    bias2 = bias_gat.reshape(1, _FEA)
    out = pl.pallas_call(
        _gat_proto_kernel,
        grid=(_B,),
        in_specs=[
            pl.BlockSpec((_N, _IN_DIM), lambda b: (b, 0)),
            pl.BlockSpec((_IN_DIM, _FEA), lambda b: (0, 0)),
            pl.BlockSpec((1, _FEA), lambda b: (0, 0)),
            pl.BlockSpec((_FEA, _H * _FEA), lambda b: (0, 0)),
            pl.BlockSpec((_H, _FEA), lambda b: (0, 0)),
            pl.BlockSpec((_H, _FEA), lambda b: (0, 0)),
            pl.BlockSpec((1, _FEA), lambda b: (0, 0)),
        ],
        out_specs=pl.BlockSpec((1, 1, _FEA), lambda b: (b, 0, 0)),
        out_shape=jax.ShapeDtypeStruct((_B, 1, _FEA), jnp.float32),
    )(x2, W_lin, blin2, W_gat, att_src, att_dst, bias2)
    return out.reshape(_B, _FEA)


# speedup vs baseline: 194.6075x; 1.6393x over previous
import jax
import jax.numpy as jnp
from jax.experimental import pallas as pl

_B, _N, _IN_DIM, _FEA, _H = 8, 64, 512, 256, 8
_NB = _B * _N
_NEG_SLOPE = 0.2


def _gat_proto_kernel(x_ref, wlin_ref, blin_ref, wgat_ref, asrc_ref,
                      adst_ref, bias_ref, out_ref):
    row_head = jax.lax.broadcasted_iota(jnp.int32, (_H * _FEA, _H), 0) // _FEA
    col_head = jax.lax.broadcasted_iota(jnp.int32, (_H * _FEA, _H), 1)
    blk = row_head == col_head
    src_bd = jnp.where(blk, asrc_ref[...], 0.0)
    dst_bd = jnp.where(blk, adst_ref[...], 0.0)
    vs = jnp.dot(wgat_ref[...], src_bd, preferred_element_type=jnp.float32)
    vd = jnp.dot(wgat_ref[...], dst_bd, preferred_element_type=jnp.float32)

    h = jnp.dot(x_ref[...], wlin_ref[...],
                preferred_element_type=jnp.float32) + blin_ref[...]
    a_s = jnp.dot(h, vs, preferred_element_type=jnp.float32)
    a_d = jnp.dot(h, vd, preferred_element_type=jnp.float32)
    a_s_t = a_s.T

    out_rows = []
    for b in range(_B):
        h_b = h[b * _N:(b + 1) * _N, :]
        w_rows = []
        for hh in range(_H):
            ad = a_d[b * _N:(b + 1) * _N, hh:hh + 1]
            asr = a_s_t[hh:hh + 1, b * _N:(b + 1) * _N]
            s = ad + asr
            s = jnp.where(s >= 0, s, _NEG_SLOPE * s)
            m = jnp.max(s, axis=1, keepdims=True)
            ex = jnp.exp(s - m)
            denom = jnp.sum(ex, axis=1, keepdims=True) + 1e-16
            w_rows.append(jnp.sum(ex / denom, axis=0, keepdims=True))
        w_mat = jnp.concatenate(w_rows, axis=0)
        g = jnp.dot(w_mat, h_b, preferred_element_type=jnp.float32)
        acc = jnp.zeros((1, _FEA), dtype=jnp.float32)
        for hh in range(_H):
            acc = acc + jnp.dot(g[hh:hh + 1, :],
                                wgat_ref[:, hh * _FEA:(hh + 1) * _FEA],
                                preferred_element_type=jnp.float32)
        out_rows.append(acc)
    out = jnp.concatenate(out_rows, axis=0)
    out_ref[...] = out * (1.0 / (_N * _H)) + bias_ref[...]


def kernel(x, W_lin, b_lin, W_gat, att_src, att_dst, bias_gat):
    x2 = x.reshape(_NB, _IN_DIM)
    blin2 = b_lin.reshape(1, _FEA)
    bias2 = bias_gat.reshape(1, _FEA)
    asrc_col = att_src.reshape(_H * _FEA, 1)
    adst_col = att_dst.reshape(_H * _FEA, 1)
    out = pl.pallas_call(
        _gat_proto_kernel,
        in_specs=[
            pl.BlockSpec((_NB, _IN_DIM), lambda: (0, 0)),
            pl.BlockSpec((_IN_DIM, _FEA), lambda: (0, 0)),
            pl.BlockSpec((1, _FEA), lambda: (0, 0)),
            pl.BlockSpec((_FEA, _H * _FEA), lambda: (0, 0)),
            pl.BlockSpec((_H * _FEA, 1), lambda: (0, 0)),
            pl.BlockSpec((_H * _FEA, 1), lambda: (0, 0)),
            pl.BlockSpec((1, _FEA), lambda: (0, 0)),
        ],
        out_specs=pl.BlockSpec((_B, _FEA), lambda: (0, 0)),
        out_shape=jax.ShapeDtypeStruct((_B, _FEA), jnp.float32),
    )(x2, W_lin, blin2, W_gat, asrc_col, adst_col, bias2)
    return out


# speedup vs baseline: 379.4210x; 1.9497x over previous
import jax
import jax.numpy as jnp
from jax.experimental import pallas as pl

_B, _N, _IN_DIM, _FEA, _H = 8, 64, 512, 256, 8
_NB = _B * _N
_NEG_SLOPE = 0.2


def _gat_proto_kernel(x_ref, wlin_ref, blin_ref, wgat_ref, asrc_ref,
                      adst_ref, bias_ref, out_ref):
    cat_t = jnp.concatenate([asrc_ref[...], adst_ref[...]], axis=0).T
    tiled = jnp.concatenate([cat_t] * _H, axis=0)
    row_head = jax.lax.broadcasted_iota(
        jnp.int32, (_H * _FEA, 2 * _H), 0) // _FEA
    col_head = jax.lax.broadcasted_iota(
        jnp.int32, (_H * _FEA, 2 * _H), 1) % _H
    bd = jnp.where(row_head == col_head, tiled, 0.0)
    vsd = jnp.dot(wgat_ref[...], bd, preferred_element_type=jnp.float32)

    h = jnp.dot(x_ref[...], wlin_ref[...],
                preferred_element_type=jnp.float32) + blin_ref[...]
    aa = jnp.dot(h, vsd, preferred_element_type=jnp.float32)
    aa_t = aa.T

    lane = jax.lax.broadcasted_iota(jnp.int32, (1, 2 * _N), 1)
    left = lane < _N

    out_rows = []
    for b in range(_B):
        sl = slice(b * _N, (b + 1) * _N)
        h_b = h[sl, :]
        w_pairs = []
        for p in range(_H // 2):
            h0, h1 = 2 * p, 2 * p + 1
            asr = jnp.concatenate(
                [aa_t[h0:h0 + 1, sl], aa_t[h1:h1 + 1, sl]], axis=1)
            ad0 = aa[sl, _H + h0:_H + h0 + 1]
            ad1 = aa[sl, _H + h1:_H + h1 + 1]
            s = jnp.where(left, ad0, ad1) + asr
            s = jnp.where(s >= 0, s, _NEG_SLOPE * s)
            m = jnp.max(s, axis=1, keepdims=True)
            ex = jnp.exp(s - m)
            d0 = jnp.sum(ex[:, :_N], axis=1, keepdims=True)
            d1 = jnp.sum(ex[:, _N:], axis=1, keepdims=True)
            inv2 = jnp.where(left, 1.0 / (d0 + 1e-16), 1.0 / (d1 + 1e-16))
            w_pairs.append(jnp.sum(ex * inv2, axis=0, keepdims=True))
        w_pack = jnp.concatenate(w_pairs, axis=0)
        g_even = jnp.dot(w_pack[:, :_N], h_b,
                         preferred_element_type=jnp.float32)
        g_odd = jnp.dot(w_pack[:, _N:], h_b,
                        preferred_element_type=jnp.float32)
        acc = jnp.zeros((1, _FEA), dtype=jnp.float32)
        for p in range(_H // 2):
            h0, h1 = 2 * p, 2 * p + 1
            acc = acc + jnp.dot(g_even[p:p + 1, :],
                                wgat_ref[:, h0 * _FEA:(h0 + 1) * _FEA],
                                preferred_element_type=jnp.float32)
            acc = acc + jnp.dot(g_odd[p:p + 1, :],
                                wgat_ref[:, h1 * _FEA:(h1 + 1) * _FEA],
                                preferred_element_type=jnp.float32)
        out_rows.append(acc)
    out = jnp.concatenate(out_rows, axis=0)
    out_ref[...] = out * (1.0 / (_N * _H)) + bias_ref[...]


def kernel(x, W_lin, b_lin, W_gat, att_src, att_dst, bias_gat):
    x2 = x.reshape(_NB, _IN_DIM)
    blin2 = b_lin.reshape(1, _FEA)
    bias2 = bias_gat.reshape(1, _FEA)
    out = pl.pallas_call(
        _gat_proto_kernel,
        in_specs=[
            pl.BlockSpec((_NB, _IN_DIM), lambda: (0, 0)),
            pl.BlockSpec((_IN_DIM, _FEA), lambda: (0, 0)),
            pl.BlockSpec((1, _FEA), lambda: (0, 0)),
            pl.BlockSpec((_FEA, _H * _FEA), lambda: (0, 0)),
            pl.BlockSpec((_H, _FEA), lambda: (0, 0)),
            pl.BlockSpec((_H, _FEA), lambda: (0, 0)),
            pl.BlockSpec((1, _FEA), lambda: (0, 0)),
        ],
        out_specs=pl.BlockSpec((_B, _FEA), lambda: (0, 0)),
        out_shape=jax.ShapeDtypeStruct((_B, _FEA), jnp.float32),
    )(x2, W_lin, blin2, W_gat, att_src, att_dst, bias2)
    return out
